# baseline (speedup 1.0000x reference)
import jax
import jax.numpy as jnp
from jax import lax
from jax.experimental import pallas as pl
from jax.experimental.pallas import tpu as pltpu
from jax.experimental.pallas import tpu_sc as plsc

BATCH = 4096
SEQ = 200
NUM_EMB = 1000000
EMBEDDING_DIM = 64
PAD_DIM = 128
LANES = 16
NUM_INDICES = BATCH * SEQ
NUM_CORES = 2
NUM_SUBCORES = 16
NUM_WORKERS = NUM_CORES * NUM_SUBCORES
PER_WORKER = NUM_INDICES // NUM_WORKERS
CHUNK = 128
NBUF = 4
HBUF = 2
ROUNDS = PER_WORKER // (CHUNK * NBUF)


PAD_BLOCK = 2000


def _pad_table(embedding):
    def body(in_ref, out_ref):
        out_ref[:, :EMBEDDING_DIM] = in_ref[...]
        out_ref[:, EMBEDDING_DIM:] = jnp.zeros(
            (PAD_BLOCK, PAD_DIM - EMBEDDING_DIM), jnp.float32
        )

    return pl.pallas_call(
        body,
        grid=(NUM_EMB // PAD_BLOCK,),
        in_specs=[
            pl.BlockSpec((PAD_BLOCK, EMBEDDING_DIM), lambda i: (i, 0)),
        ],
        out_specs=pl.BlockSpec((PAD_BLOCK, PAD_DIM), lambda i: (i, 0)),
        out_shape=jax.ShapeDtypeStruct((NUM_EMB, PAD_DIM), jnp.float32),
    )(embedding)


def kernel(token_ids, embedding):
    flat_ids = token_ids.reshape(NUM_INDICES)
    table128 = _pad_table(embedding)

    mesh = plsc.VectorSubcoreMesh(core_axis_name="c", subcore_axis_name="s")

    @pl.kernel(
        out_type=jax.ShapeDtypeStruct((BATCH, SEQ, EMBEDDING_DIM), embedding.dtype),
        mesh=mesh,
        scratch_types=[
            pltpu.VMEM((NBUF, CHUNK), jnp.int32),
            pltpu.VMEM((NBUF, CHUNK, PAD_DIM), jnp.float32),
            pltpu.VMEM((HBUF, CHUNK, EMBEDDING_DIM), jnp.float32),
            pltpu.SemaphoreType.DMA((NBUF,)),
            pltpu.SemaphoreType.DMA((NBUF,)),
            pltpu.SemaphoreType.DMA((HBUF,)),
        ],
    )
    def gather_kernel(
        table_hbm, idx_hbm, out3_hbm, idx_v, rows_v, half_v, sem_i, sem_g, sem_o
    ):
        out_hbm = out3_hbm.reshape(NUM_INDICES, EMBEDDING_DIM)
        wid = lax.axis_index("s") * NUM_CORES + lax.axis_index("c")
        base = wid * PER_WORKER

        for b in range(NBUF):
            pltpu.async_copy(
                idx_hbm.at[pl.ds(base + b * CHUNK, CHUNK)], idx_v.at[b], sem_i.at[b]
            )

        @pl.loop(0, ROUNDS)
        def _(r):
            g0 = base + r * (NBUF * CHUNK)

            for b in range(NBUF):
                pltpu.make_async_copy(
                    idx_hbm.at[pl.ds(g0 + b * CHUNK, CHUNK)],
                    idx_v.at[b],
                    sem_i.at[b],
                ).wait()
                pltpu.async_copy(
                    table_hbm.at[idx_v.at[b]], rows_v.at[b], sem_g.at[b]
                )

            for b in range(NBUF):
                start = g0 + b * CHUNK
                h = b % HBUF
                pltpu.make_async_copy(
                    table_hbm.at[idx_v.at[b]], rows_v.at[b], sem_g.at[b]
                ).wait()

                def drain():
                    pltpu.make_async_copy(
                        half_v.at[h],
                        out_hbm.at[pl.ds(base, CHUNK)],
                        sem_o.at[h],
                    ).wait()

                if b >= HBUF:
                    drain()
                else:
                    pl.when(r > 0)(drain)

                @pl.loop(0, CHUNK)
                def _(j):
                    for c in range(0, EMBEDDING_DIM, LANES):
                        half_v[h, j, pl.ds(c, LANES)] = rows_v[b, j, pl.ds(c, LANES)]

                pltpu.async_copy(
                    half_v.at[h], out_hbm.at[pl.ds(start, CHUNK)], sem_o.at[h]
                )

                @pl.when(r + 1 < ROUNDS)
                def _():
                    pltpu.async_copy(
                        idx_hbm.at[pl.ds(g0 + (NBUF + b) * CHUNK, CHUNK)],
                        idx_v.at[b],
                        sem_i.at[b],
                    )

        for h in range(HBUF):
            pltpu.make_async_copy(
                half_v.at[h], out_hbm.at[pl.ds(base, CHUNK)], sem_o.at[h]
            ).wait()

    return gather_kernel(table128, flat_ids)


# speedup vs baseline: 1.2408x; 1.2408x over previous
import jax
import jax.numpy as jnp
from jax import lax
from jax.experimental import pallas as pl
from jax.experimental.pallas import tpu as pltpu
from jax.experimental.pallas import tpu_sc as plsc

BATCH = 4096
SEQ = 200
NUM_EMB = 1000000
EMBEDDING_DIM = 64
PAD_DIM = 128
LANES = 16
NUM_INDICES = BATCH * SEQ
NUM_CORES = 2
NUM_SUBCORES = 16
NUM_WORKERS = NUM_CORES * NUM_SUBCORES
BATCHES_PER_WORKER = BATCH // NUM_WORKERS
CHUNK_A = 128
CHUNK_B = SEQ - CHUNK_A
PAIRS = BATCHES_PER_WORKER // 2


def kernel(token_ids, embedding):
    table128 = jnp.pad(embedding, ((0, 0), (0, PAD_DIM - EMBEDDING_DIM)))

    mesh = plsc.VectorSubcoreMesh(core_axis_name="c", subcore_axis_name="s")

    @pl.kernel(
        out_type=jax.ShapeDtypeStruct((BATCH, SEQ, EMBEDDING_DIM), embedding.dtype),
        mesh=mesh,
        scratch_types=[
            pltpu.VMEM((2, SEQ), jnp.int32),
            pltpu.VMEM((2, CHUNK_A, PAD_DIM), jnp.float32),
            pltpu.VMEM((2, CHUNK_B, PAD_DIM), jnp.float32),
            pltpu.VMEM((2, CHUNK_A, EMBEDDING_DIM), jnp.float32),
            pltpu.VMEM((2, CHUNK_B, EMBEDDING_DIM), jnp.float32),
            pltpu.SemaphoreType.DMA((2,)),
            pltpu.SemaphoreType.DMA((2,)),
            pltpu.SemaphoreType.DMA((2,)),
            pltpu.SemaphoreType.DMA((2,)),
            pltpu.SemaphoreType.DMA((2,)),
        ],
    )
    def gather_kernel(
        table_hbm, idx_hbm, out_hbm,
        idx_v, rows_a, rows_b, half_a, half_b,
        sem_i, sem_ga, sem_gb, sem_oa, sem_ob,
    ):
        wid = lax.axis_index("s") * NUM_CORES + lax.axis_index("c")
        base_b = wid * BATCHES_PER_WORKER

        def start_idx(r):
            for k in range(2):
                b = base_b + 2 * r + k
                pltpu.async_copy(idx_hbm.at[b], idx_v.at[k], sem_i.at[k])

        def wait_idx():
            for k in range(2):
                pltpu.make_async_copy(
                    idx_hbm.at[0], idx_v.at[k], sem_i.at[k]
                ).wait()

        def compact(dst, src, n):
            @pl.loop(0, n)
            def _(j):
                for c in range(0, EMBEDDING_DIM, LANES):
                    dst[j, pl.ds(c, LANES)] = src[j, pl.ds(c, LANES)]

        start_idx(0)

        @pl.loop(0, PAIRS)
        def _(r):
            wait_idx()

            for k in range(2):
                def drain_outs(k=k):
                    pltpu.make_async_copy(
                        half_a.at[k], out_hbm.at[0, pl.ds(0, CHUNK_A)], sem_oa.at[k]
                    ).wait()
                    pltpu.make_async_copy(
                        half_b.at[k], out_hbm.at[0, pl.ds(0, CHUNK_B)], sem_ob.at[k]
                    ).wait()

                pl.when(r > 0)(drain_outs)
                pltpu.async_copy(
                    table_hbm.at[idx_v.at[k, pl.ds(0, CHUNK_A)]],
                    rows_a.at[k],
                    sem_ga.at[k],
                )
                pltpu.async_copy(
                    table_hbm.at[idx_v.at[k, pl.ds(CHUNK_A, CHUNK_B)]],
                    rows_b.at[k],
                    sem_gb.at[k],
                )

            for k in range(2):
                b = base_b + 2 * r + k
                pltpu.make_async_copy(
                    table_hbm.at[idx_v.at[k, pl.ds(0, CHUNK_A)]],
                    rows_a.at[k],
                    sem_ga.at[k],
                ).wait()
                compact(half_a.at[k], rows_a.at[k], CHUNK_A)
                pltpu.async_copy(
                    half_a.at[k], out_hbm.at[b, pl.ds(0, CHUNK_A)], sem_oa.at[k]
                )
                pltpu.make_async_copy(
                    table_hbm.at[idx_v.at[k, pl.ds(CHUNK_A, CHUNK_B)]],
                    rows_b.at[k],
                    sem_gb.at[k],
                ).wait()
                compact(half_b.at[k], rows_b.at[k], CHUNK_B)
                pltpu.async_copy(
                    half_b.at[k], out_hbm.at[b, pl.ds(CHUNK_A, CHUNK_B)], sem_ob.at[k]
                )

            @pl.when(r + 1 < PAIRS)
            def _():
                start_idx(r + 1)

        for k in range(2):
            pltpu.make_async_copy(
                half_a.at[k], out_hbm.at[0, pl.ds(0, CHUNK_A)], sem_oa.at[k]
            ).wait()
            pltpu.make_async_copy(
                half_b.at[k], out_hbm.at[0, pl.ds(0, CHUNK_B)], sem_ob.at[k]
            ).wait()

    return gather_kernel(table128, token_ids)


# speedup vs baseline: 1.4145x; 1.1400x over previous
import jax
import jax.numpy as jnp
from jax import lax
from jax.experimental import pallas as pl
from jax.experimental.pallas import tpu as pltpu
from jax.experimental.pallas import tpu_sc as plsc

BATCH = 4096
SEQ = 200
NUM_EMB = 1000000
EMBEDDING_DIM = 64
PAD_DIM = 128
LANES = 16
NUM_INDICES = BATCH * SEQ
NUM_CORES = 2
NUM_SUBCORES = 16
NUM_WORKERS = NUM_CORES * NUM_SUBCORES
PER_WORKER = NUM_INDICES // NUM_WORKERS
CHUNK = 128
NBUF = 5
HBUF = 2
ROUNDS = PER_WORKER // (CHUNK * NBUF)


def kernel(token_ids, embedding):
    flat_ids = token_ids.reshape(NUM_INDICES)
    table128 = jnp.pad(embedding, ((0, 0), (0, PAD_DIM - EMBEDDING_DIM)))

    mesh = plsc.VectorSubcoreMesh(core_axis_name="c", subcore_axis_name="s")

    @pl.kernel(
        out_type=jax.ShapeDtypeStruct((NUM_INDICES, EMBEDDING_DIM), embedding.dtype),
        mesh=mesh,
        scratch_types=[
            pltpu.VMEM((NBUF, CHUNK), jnp.int32),
            pltpu.VMEM((NBUF, CHUNK, PAD_DIM), jnp.float32),
            pltpu.VMEM((HBUF, CHUNK, EMBEDDING_DIM), jnp.float32),
            pltpu.SemaphoreType.DMA((NBUF,)),
            pltpu.SemaphoreType.DMA((NBUF,)),
            pltpu.SemaphoreType.DMA((HBUF,)),
        ],
    )
    def gather_kernel(
        table_hbm, idx_hbm, out_hbm, idx_v, rows_v, half_v, sem_i, sem_g, sem_o
    ):
        wid = lax.axis_index("s") * NUM_CORES + lax.axis_index("c")
        base = wid * PER_WORKER

        for b in range(NBUF):
            pltpu.async_copy(
                idx_hbm.at[pl.ds(base + b * CHUNK, CHUNK)], idx_v.at[b], sem_i.at[b]
            )

        @pl.loop(0, ROUNDS)
        def _(r):
            g0 = base + r * (NBUF * CHUNK)

            for b in range(NBUF):
                pltpu.make_async_copy(
                    idx_hbm.at[pl.ds(g0 + b * CHUNK, CHUNK)],
                    idx_v.at[b],
                    sem_i.at[b],
                ).wait()
                pltpu.async_copy(
                    table_hbm.at[idx_v.at[b]], rows_v.at[b], sem_g.at[b]
                )

            for b in range(NBUF):
                start = g0 + b * CHUNK
                h = b % HBUF
                pltpu.make_async_copy(
                    table_hbm.at[idx_v.at[b]], rows_v.at[b], sem_g.at[b]
                ).wait()

                def drain():
                    pltpu.make_async_copy(
                        half_v.at[h],
                        out_hbm.at[pl.ds(base, CHUNK)],
                        sem_o.at[h],
                    ).wait()

                if b >= HBUF:
                    drain()
                else:
                    pl.when(r > 0)(drain)

                @pl.loop(0, CHUNK)
                def _(j):
                    for c in range(0, EMBEDDING_DIM, LANES):
                        half_v[h, j, pl.ds(c, LANES)] = rows_v[b, j, pl.ds(c, LANES)]

                pltpu.async_copy(
                    half_v.at[h], out_hbm.at[pl.ds(start, CHUNK)], sem_o.at[h]
                )

                @pl.when(r + 1 < ROUNDS)
                def _():
                    pltpu.async_copy(
                        idx_hbm.at[pl.ds(g0 + (NBUF + b) * CHUNK, CHUNK)],
                        idx_v.at[b],
                        sem_i.at[b],
                    )

        for h in range(HBUF):
            pltpu.make_async_copy(
                half_v.at[h], out_hbm.at[pl.ds(base, CHUNK)], sem_o.at[h]
            ).wait()

    out = gather_kernel(table128, flat_ids)
    return out.reshape(BATCH, SEQ, EMBEDDING_DIM)
